# Initial kernel scaffold; baseline (speedup 1.0000x reference)
import jax
import jax.numpy as jnp
from jax.experimental import pallas as pl


def kernel(x, edge_index, Wp0, bp0, Ws0, Wn0, b0, Wp1, bp1, Ws1, Wn1, b1):
    raise NotImplementedError("write your pallas kernel here")


# speedup vs baseline: 1.6924x; 1.6924x over previous
import functools

import jax
import jax.numpy as jnp
from jax import lax
from jax.experimental import pallas as pl
from jax.experimental.pallas import tpu as pltpu
from jax.experimental.pallas import tpu_sc as plsc

F32 = jnp.float32
I32 = jnp.int32

NW = 32
L = 16
D = 128
CHUNK = 2000
GB = 128


def _segmax_sc(hp, src, dst, n_nodes):
    E = src.shape[0]
    R = (-(-n_nodes // NW) + 7) // 8 * 8
    NPAD = NW * R
    RLOC = R + 8
    DUMP = R
    NCHUNK = E // CHUNK
    assert NCHUNK * CHUNK == E
    NGRP = CHUNK // L
    MBUF = CHUNK + GB + L

    mesh = plsc.VectorSubcoreMesh(core_axis_name="c", subcore_axis_name="s")

    @functools.partial(
        pl.kernel,
        out_type=jax.ShapeDtypeStruct((NPAD, D), F32),
        mesh=mesh,
        scratch_types=[
            pltpu.VMEM((CHUNK,), I32),
            pltpu.VMEM((CHUNK,), I32),
            pltpu.VMEM((MBUF,), I32),
            pltpu.VMEM((MBUF,), I32),
            pltpu.VMEM((GB, D), F32),
            pltpu.VMEM((RLOC, D), F32),
            pltpu.SemaphoreType.DMA,
        ],
        compiler_params=pltpu.CompilerParams(needs_layout_passes=False),
    )
    def k(hp_hbm, src_hbm, dst_hbm, out_hbm, src_v, dst_v, msrc, mdst,
          rows, acc, sem):
        wid = lax.axis_index("s") * 2 + lax.axis_index("c")
        lo = wid * R
        zero16 = jnp.zeros((L,), F32)
        lane = lax.iota(I32, L)

        @pl.loop(0, RLOC)
        def _(r):
            for c in range(D // L):
                acc[r, pl.ds(c * L, L)] = zero16

        def process_batch(off):
            pltpu.async_copy(hp_hbm.at[msrc.at[pl.ds(off, GB)]], rows,
                             sem).wait()

            @pl.loop(0, GB // L)
            def _(g):
                dv = mdst[pl.ds(off + g * L, L)]
                for l in range(L):
                    d = jnp.max(jnp.where(lane == l, dv, 0))
                    j = g * L + l
                    for c in range(D // L):
                        sl = pl.ds(c * L, L)
                        acc[d, sl] = jnp.maximum(acc[d, sl], rows[j, sl])

        def scan_chunk(ci, pos):
            pltpu.sync_copy(src_hbm.at[pl.ds(ci * CHUNK, CHUNK)], src_v)
            pltpu.sync_copy(dst_hbm.at[pl.ds(ci * CHUNK, CHUNK)], dst_v)

            def group(gi, pos):
                s16 = src_v[pl.ds(gi * L, L)]
                dl = dst_v[pl.ds(gi * L, L)] - lo
                mask = (dl >= 0) & (dl < R)
                pm = plsc.cumsum(mask.astype(I32))
                idx = pos + pm - 1
                plsc.store_scatter(msrc, [idx], s16, mask=mask)
                plsc.store_scatter(mdst, [idx], dl, mask=mask)
                return pos + jnp.max(pm)

            pos = lax.fori_loop(0, NGRP, group, pos)
            nb = pos // GB

            def run(b, carry):
                process_batch(b * GB)
                return carry

            lax.fori_loop(0, nb, run, 0)
            base = nb * GB
            for t in range(GB // L):
                msrc[pl.ds(t * L, L)] = msrc[pl.ds(base + t * L, L)]
                mdst[pl.ds(t * L, L)] = mdst[pl.ds(base + t * L, L)]
            return pos - base

        pos = lax.fori_loop(0, NCHUNK, scan_chunk, 0)

        zi16 = jnp.zeros((L,), I32)
        dump16 = jnp.full((L,), DUMP, I32)
        for t in range(GB // L):
            idxp = pos + t * L + lane
            plsc.store_scatter(msrc, [idxp], zi16)
            plsc.store_scatter(mdst, [idxp], dump16)
        nb = (pos + GB - 1) // GB

        def run(b, carry):
            process_batch(b * GB)
            return carry

        lax.fori_loop(0, nb, run, 0)

        pltpu.sync_copy(acc.at[pl.ds(0, R)], out_hbm.at[pl.ds(lo, R)])

    return k(hp, src, dst)


def _lin_relu(x, W, b):
    M, K = x.shape
    Kn = W.shape[1]
    BM = 1000

    def body(x_ref, w_ref, b_ref, o_ref):
        o_ref[...] = jnp.maximum(
            jnp.dot(x_ref[...], w_ref[...], preferred_element_type=F32)
            + b_ref[...], 0.0)

    return pl.pallas_call(
        body,
        grid=(M // BM,),
        in_specs=[
            pl.BlockSpec((BM, K), lambda i: (i, 0)),
            pl.BlockSpec((K, Kn), lambda i: (0, 0)),
            pl.BlockSpec((1, Kn), lambda i: (0, 0)),
        ],
        out_specs=pl.BlockSpec((BM, Kn), lambda i: (i, 0)),
        out_shape=jax.ShapeDtypeStruct((M, Kn), F32),
    )(x, W, b.reshape(1, Kn))


def _mid(x, hn, Ws, Wn, b, Wp, bp):
    M, K = x.shape
    BM = 1000

    def body(x_ref, hn_ref, ws_ref, wn_ref, b_ref, wp_ref, bp_ref,
             h1_ref, hp1_ref):
        t = (jnp.dot(x_ref[...], ws_ref[...], preferred_element_type=F32)
             + jnp.dot(hn_ref[...], wn_ref[...], preferred_element_type=F32)
             + b_ref[...])
        t = jnp.maximum(t, 0.0)
        nrm = jnp.sqrt(jnp.sum(t * t, axis=1, keepdims=True))
        h1 = t / jnp.maximum(nrm, 1e-12)
        h1_ref[...] = h1
        hp1_ref[...] = jnp.maximum(
            jnp.dot(h1, wp_ref[...], preferred_element_type=F32)
            + bp_ref[...], 0.0)

    return pl.pallas_call(
        body,
        grid=(M // BM,),
        in_specs=[
            pl.BlockSpec((BM, K), lambda i: (i, 0)),
            pl.BlockSpec((BM, K), lambda i: (i, 0)),
            pl.BlockSpec((K, K), lambda i: (0, 0)),
            pl.BlockSpec((K, K), lambda i: (0, 0)),
            pl.BlockSpec((1, K), lambda i: (0, 0)),
            pl.BlockSpec((K, K), lambda i: (0, 0)),
            pl.BlockSpec((1, K), lambda i: (0, 0)),
        ],
        out_specs=[
            pl.BlockSpec((BM, K), lambda i: (i, 0)),
            pl.BlockSpec((BM, K), lambda i: (i, 0)),
        ],
        out_shape=[
            jax.ShapeDtypeStruct((M, K), F32),
            jax.ShapeDtypeStruct((M, K), F32),
        ],
    )(x, hn, Ws, Wn, b.reshape(1, K), Wp, bp.reshape(1, K))


def _final(h1, hn, Ws, Wn, b):
    M, K = h1.shape
    C = Ws.shape[1]

    Wsp = jnp.zeros((K, D), F32).at[:, :C].set(Ws)
    Wnp = jnp.zeros((K, D), F32).at[:, :C].set(Wn)
    bp = jnp.zeros((1, D), F32).at[0, :C].set(b)
    BM = 1000

    def body(h_ref, hn_ref, ws_ref, wn_ref, b_ref, o_ref):
        o_ref[...] = (
            jnp.dot(h_ref[...], ws_ref[...], preferred_element_type=F32)
            + jnp.dot(hn_ref[...], wn_ref[...], preferred_element_type=F32)
            + b_ref[...])

    out = pl.pallas_call(
        body,
        grid=(M // BM,),
        in_specs=[
            pl.BlockSpec((BM, K), lambda i: (i, 0)),
            pl.BlockSpec((BM, K), lambda i: (i, 0)),
            pl.BlockSpec((K, D), lambda i: (0, 0)),
            pl.BlockSpec((K, D), lambda i: (0, 0)),
            pl.BlockSpec((1, D), lambda i: (0, 0)),
        ],
        out_specs=pl.BlockSpec((BM, D), lambda i: (i, 0)),
        out_shape=jax.ShapeDtypeStruct((M, D), F32),
    )(h1, hn, Wsp, Wnp, bp)
    return out[:, :C]


def kernel(x, edge_index, Wp0, bp0, Ws0, Wn0, b0, Wp1, bp1, Ws1, Wn1, b1):
    n = x.shape[0]
    src = edge_index[0].astype(I32)
    dst = edge_index[1].astype(I32)

    hp0 = _lin_relu(x, Wp0, bp0)
    hn0 = _segmax_sc(hp0, src, dst, n)[:n]
    h1, hp1 = _mid(x, hn0, Ws0, Wn0, b0, Wp1, bp1)
    hn1 = _segmax_sc(hp1, src, dst, n)[:n]
    return _final(h1, hn1, Ws1, Wn1, b1)


# speedup vs baseline: 3.1738x; 1.8753x over previous
import functools

import jax
import jax.numpy as jnp
from jax import lax
from jax.experimental import pallas as pl
from jax.experimental.pallas import tpu as pltpu
from jax.experimental.pallas import tpu_sc as plsc

F32 = jnp.float32
I32 = jnp.int32

NW = 32
L = 16
D = 128
CHUNK = 2000
GB = 128

_MESH = plsc.VectorSubcoreMesh(core_axis_name="c", subcore_axis_name="s")
_SC_PARAMS = pltpu.CompilerParams(needs_layout_passes=False)
FLUSH = 256
SHIFT = 14


def _geom(n_nodes, E):
    R = (-(-n_nodes // NW) + 7) // 8 * 8
    WSTRIDE = E + FLUSH
    return R, NW * R, WSTRIDE


def _prep_sc(src, dst, n_nodes):
    E = src.shape[0]
    R, _, WSTRIDE = _geom(n_nodes, E)
    DUMP = R
    NCHUNK = E // CHUNK
    assert NCHUNK * CHUNK == E and NCHUNK % 2 == 0
    NGRP = CHUNK // L
    MBUF = CHUNK + FLUSH + L

    @functools.partial(
        pl.kernel,
        out_type=(jax.ShapeDtypeStruct((NW * WSTRIDE,), I32),
                  jax.ShapeDtypeStruct((NW * L,), I32)),
        mesh=_MESH,
        scratch_types=[
            pltpu.VMEM((CHUNK,), I32), pltpu.VMEM((CHUNK,), I32),
            pltpu.VMEM((CHUNK,), I32), pltpu.VMEM((CHUNK,), I32),
            pltpu.VMEM((MBUF,), I32),
            pltpu.VMEM((FLUSH,), I32), pltpu.VMEM((FLUSH,), I32),
            pltpu.VMEM((L,), I32),
            pltpu.SemaphoreType.DMA, pltpu.SemaphoreType.DMA,
            pltpu.SemaphoreType.DMA, pltpu.SemaphoreType.DMA,
        ],
        compiler_params=_SC_PARAMS,
    )
    def k(src_hbm, dst_hbm, plist_hbm, cnt_hbm, src_v0, src_v1, dst_v0,
          dst_v1, mbuf, fstage0, fstage1, cstage, sem_in0, sem_in1,
          sem_f0, sem_f1):
        wid = lax.axis_index("s") * 2 + lax.axis_index("c")
        lo = wid * R
        base_out = wid * WSTRIDE
        lane = lax.iota(I32, L)
        srcs = (src_v0, src_v1)
        dsts = (dst_v0, dst_v1)
        insems = (sem_in0, sem_in1)
        fstgs = (fstage0, fstage1)
        fsems = (sem_f0, sem_f1)

        def load_chunk(ci, s):
            pltpu.async_copy(src_hbm.at[pl.ds(ci * CHUNK, CHUNK)],
                             srcs[s], insems[s])
            pltpu.async_copy(dst_hbm.at[pl.ds(ci * CHUNK, CHUNK)],
                             dsts[s], insems[s])

        def wait_chunk(s):
            pltpu.make_async_copy(src_hbm.at[pl.ds(0, CHUNK)],
                                  srcs[s], insems[s]).wait()
            pltpu.make_async_copy(dst_hbm.at[pl.ds(0, CHUNK)],
                                  dsts[s], insems[s]).wait()

        def flush(pos, nf):
            def flush_slot(s):
                stg = fstgs[s]

                @pl.when(nf >= 2)
                def _():
                    pltpu.make_async_copy(
                        plist_hbm.at[pl.ds(0, FLUSH)], stg,
                        fsems[s]).wait()
                for t in range(FLUSH // L):
                    stg[pl.ds(t * L, L)] = mbuf[pl.ds(t * L, L)]
                pltpu.async_copy(
                    stg,
                    plist_hbm.at[pl.ds(base_out + nf * FLUSH, FLUSH)],
                    fsems[s])

            @pl.when(nf % 2 == 0)
            def _():
                flush_slot(0)

            @pl.when(nf % 2 == 1)
            def _():
                flush_slot(1)
            for t in range(MBUF // L - FLUSH // L):
                mbuf[pl.ds(t * L, L)] = mbuf[pl.ds(FLUSH + t * L, L)]
            return pos - FLUSH

        def scan_one(ci, s, carry):
            pos, nf = carry
            wait_chunk(s)

            def group(gi, pos):
                s16 = srcs[s][pl.ds(gi * L, L)]
                dl = dsts[s][pl.ds(gi * L, L)] - lo
                mask = (dl >= 0) & (dl < R)
                pm = plsc.cumsum(mask.astype(I32))
                idx = pos + pm - 1
                plsc.store_scatter(mbuf, [idx],
                                   s16 | (dl << SHIFT), mask=mask)
                return pos + jnp.max(pm)

            pos = lax.fori_loop(0, NGRP, group, pos)

            @pl.when(ci + 2 < NCHUNK)
            def _():
                load_chunk(ci + 2, s)

            def do_flush(_, carry):
                pos, nf = carry
                return flush(pos, nf), nf + 1

            return lax.fori_loop(0, pos // FLUSH, do_flush, (pos, nf))

        load_chunk(0, 0)
        load_chunk(1, 1)

        def scan_pair(i, carry):
            carry = scan_one(2 * i, 0, carry)
            return scan_one(2 * i + 1, 1, carry)

        pos, nf = lax.fori_loop(0, NCHUNK // 2, scan_pair, (0, 0))

        padword = (wid * 8) | (DUMP << SHIFT)
        pad16 = jnp.full((L,), padword, I32)
        for t in range(FLUSH // L):
            plsc.store_scatter(mbuf, [pos + t * L + lane], pad16)
        flush(pos, nf)
        total = nf + 1

        cstage[...] = jnp.full((L,), total * FLUSH, I32)
        pltpu.sync_copy(cstage, cnt_hbm.at[pl.ds(wid * L, L)])
        pltpu.make_async_copy(plist_hbm.at[pl.ds(0, FLUSH)], fstage0,
                              sem_f0).wait()

        @pl.when(total >= 2)
        def _():
            pltpu.make_async_copy(plist_hbm.at[pl.ds(0, FLUSH)],
                                  fstage1, sem_f1).wait()

    return k(src, dst)


def _segmax_sc(hp, plist, counts, n_nodes, E):
    R, NPAD, WSTRIDE = _geom(n_nodes, E)
    RLOC = R + 8

    @functools.partial(
        pl.kernel,
        out_type=jax.ShapeDtypeStruct((NPAD, D), F32),
        mesh=_MESH,
        scratch_types=[
            pltpu.VMEM((GB,), I32), pltpu.VMEM((GB,), I32),
            pltpu.VMEM((GB,), I32), pltpu.VMEM((GB,), I32),
            pltpu.VMEM((GB,), I32), pltpu.VMEM((GB,), I32),
            pltpu.VMEM((GB, D), F32), pltpu.VMEM((GB, D), F32),
            pltpu.VMEM((RLOC, D), F32),
            pltpu.VMEM((L,), I32),
            pltpu.SemaphoreType.DMA, pltpu.SemaphoreType.DMA,
            pltpu.SemaphoreType.DMA, pltpu.SemaphoreType.DMA,
        ],
        compiler_params=_SC_PARAMS,
    )
    def k(hp_hbm, plist_hbm, cnt_hbm, out_hbm, pbuf0, pbuf1, sidx0,
          sidx1, didx0, didx1, rows0, rows1, acc, cntv, sem_i0, sem_i1,
          sem_g0, sem_g1):
        wid = lax.axis_index("s") * 2 + lax.axis_index("c")
        lo = wid * R
        base = wid * WSTRIDE
        zero16 = jnp.zeros((L,), F32)
        lane = lax.iota(I32, L)
        pbufs = (pbuf0, pbuf1)
        sidxs = (sidx0, sidx1)
        didxs = (didx0, didx1)
        rowss = (rows0, rows1)
        isems = (sem_i0, sem_i1)
        gsems = (sem_g0, sem_g1)

        @pl.loop(0, RLOC)
        def _(r):
            for c in range(D // L):
                acc[r, pl.ds(c * L, L)] = zero16

        pltpu.sync_copy(cnt_hbm.at[pl.ds(wid * L, L)], cntv)
        nb = jnp.max(cntv[...]) // GB

        def load_p(b, s):
            off = jnp.minimum(b, nb - 1) * GB
            pltpu.async_copy(plist_hbm.at[pl.ds(base + off, GB)],
                             pbufs[s], isems[s])

        def wait_p(s):
            pltpu.make_async_copy(plist_hbm.at[pl.ds(0, GB)], pbufs[s],
                                  isems[s]).wait()

        def decode(s):
            for t in range(GB // L):
                w = pbufs[s][pl.ds(t * L, L)]
                sidxs[s][pl.ds(t * L, L)] = w & ((1 << SHIFT) - 1)
                didxs[s][pl.ds(t * L, L)] = lax.shift_right_logical(
                    w, SHIFT)

        def gather(s):
            pltpu.async_copy(hp_hbm.at[sidxs[s]], rowss[s], gsems[s])

        def wait_g(s):
            pltpu.make_async_copy(hp_hbm.at[sidxs[s]], rowss[s],
                                  gsems[s]).wait()

        def rmw(s):
            @pl.loop(0, GB // L)
            def _(g):
                dv = didxs[s][pl.ds(g * L, L)]
                for l in range(L):
                    d = jnp.max(jnp.where(lane == l, dv, 0))
                    j = g * L + l
                    for c in range(D // L):
                        sl = pl.ds(c * L, L)
                        acc[d, sl] = jnp.maximum(acc[d, sl],
                                                 rowss[s][j, sl])

        pltpu.sync_copy(plist_hbm.at[pl.ds(base, GB)], pbuf0)
        decode(0)
        gather(0)
        load_p(1, 1)

        def step(i, carry):
            b = 2 * i
            wait_p(1)
            decode(1)
            load_p(b + 2, 0)
            wait_g(0)
            gather(1)
            rmw(0)
            wait_p(0)
            decode(0)
            load_p(b + 3, 1)
            wait_g(1)
            gather(0)
            rmw(1)
            return carry

        lax.fori_loop(0, nb // 2, step, 0)
        pltpu.make_async_copy(plist_hbm.at[pl.ds(0, GB)], pbuf1,
                              sem_i1).wait()
        wait_g(0)

        pltpu.sync_copy(acc.at[pl.ds(0, R)], out_hbm.at[pl.ds(lo, R)])

    return k(hp, plist, counts)


def _lin_relu(x, W, b):
    M, K = x.shape
    Kn = W.shape[1]
    BM = 1000

    def body(x_ref, w_ref, b_ref, o_ref):
        o_ref[...] = jnp.maximum(
            jnp.dot(x_ref[...], w_ref[...], preferred_element_type=F32)
            + b_ref[...], 0.0)

    return pl.pallas_call(
        body,
        grid=(M // BM,),
        in_specs=[
            pl.BlockSpec((BM, K), lambda i: (i, 0)),
            pl.BlockSpec((K, Kn), lambda i: (0, 0)),
            pl.BlockSpec((1, Kn), lambda i: (0, 0)),
        ],
        out_specs=pl.BlockSpec((BM, Kn), lambda i: (i, 0)),
        out_shape=jax.ShapeDtypeStruct((M, Kn), F32),
    )(x, W, b.reshape(1, Kn))


def _mid(x, hn, Ws, Wn, b, Wp, bp):
    M, K = x.shape
    BM = 1000

    def body(x_ref, hn_ref, ws_ref, wn_ref, b_ref, wp_ref, bp_ref,
             h1_ref, hp1_ref):
        t = (jnp.dot(x_ref[...], ws_ref[...], preferred_element_type=F32)
             + jnp.dot(hn_ref[...], wn_ref[...], preferred_element_type=F32)
             + b_ref[...])
        t = jnp.maximum(t, 0.0)
        nrm = jnp.sqrt(jnp.sum(t * t, axis=1, keepdims=True))
        h1 = t / jnp.maximum(nrm, 1e-12)
        h1_ref[...] = h1
        hp1_ref[...] = jnp.maximum(
            jnp.dot(h1, wp_ref[...], preferred_element_type=F32)
            + bp_ref[...], 0.0)

    return pl.pallas_call(
        body,
        grid=(M // BM,),
        in_specs=[
            pl.BlockSpec((BM, K), lambda i: (i, 0)),
            pl.BlockSpec((BM, K), lambda i: (i, 0)),
            pl.BlockSpec((K, K), lambda i: (0, 0)),
            pl.BlockSpec((K, K), lambda i: (0, 0)),
            pl.BlockSpec((1, K), lambda i: (0, 0)),
            pl.BlockSpec((K, K), lambda i: (0, 0)),
            pl.BlockSpec((1, K), lambda i: (0, 0)),
        ],
        out_specs=[
            pl.BlockSpec((BM, K), lambda i: (i, 0)),
            pl.BlockSpec((BM, K), lambda i: (i, 0)),
        ],
        out_shape=[
            jax.ShapeDtypeStruct((M, K), F32),
            jax.ShapeDtypeStruct((M, K), F32),
        ],
    )(x, hn, Ws, Wn, b.reshape(1, K), Wp, bp.reshape(1, K))


def _final(h1, hn, Ws, Wn, b):
    M, K = h1.shape
    C = Ws.shape[1]

    Wsp = jnp.zeros((K, D), F32).at[:, :C].set(Ws)
    Wnp = jnp.zeros((K, D), F32).at[:, :C].set(Wn)
    bp = jnp.zeros((1, D), F32).at[0, :C].set(b)
    BM = 1000

    def body(h_ref, hn_ref, ws_ref, wn_ref, b_ref, o_ref):
        o_ref[...] = (
            jnp.dot(h_ref[...], ws_ref[...], preferred_element_type=F32)
            + jnp.dot(hn_ref[...], wn_ref[...], preferred_element_type=F32)
            + b_ref[...])

    out = pl.pallas_call(
        body,
        grid=(M // BM,),
        in_specs=[
            pl.BlockSpec((BM, K), lambda i: (i, 0)),
            pl.BlockSpec((BM, K), lambda i: (i, 0)),
            pl.BlockSpec((K, D), lambda i: (0, 0)),
            pl.BlockSpec((K, D), lambda i: (0, 0)),
            pl.BlockSpec((1, D), lambda i: (0, 0)),
        ],
        out_specs=pl.BlockSpec((BM, D), lambda i: (i, 0)),
        out_shape=jax.ShapeDtypeStruct((M, D), F32),
    )(h1, hn, Wsp, Wnp, bp)
    return out[:, :C]


def kernel(x, edge_index, Wp0, bp0, Ws0, Wn0, b0, Wp1, bp1, Ws1, Wn1, b1):
    n = x.shape[0]
    E = edge_index.shape[1]
    src = edge_index[0].astype(I32)
    dst = edge_index[1].astype(I32)

    plist, counts = _prep_sc(src, dst, n)
    hp0 = _lin_relu(x, Wp0, bp0)
    hn0 = _segmax_sc(hp0, plist, counts, n, E)[:n]
    h1, hp1 = _mid(x, hn0, Ws0, Wn0, b0, Wp1, bp1)
    hn1 = _segmax_sc(hp1, plist, counts, n, E)[:n]
    return _final(h1, hn1, Ws1, Wn1, b1)


# speedup vs baseline: 3.9424x; 1.2422x over previous
import functools

import jax
import jax.numpy as jnp
from jax import lax
from jax.experimental import pallas as pl
from jax.experimental.pallas import tpu as pltpu
from jax.experimental.pallas import tpu_sc as plsc

F32 = jnp.float32
I32 = jnp.int32
BF16 = jnp.bfloat16

NW = 32
L = 16
D = 128
CHUNK = 2000
GB = 128

_MESH = plsc.VectorSubcoreMesh(core_axis_name="c", subcore_axis_name="s")
_SC_PARAMS = pltpu.CompilerParams(needs_layout_passes=False)
FLUSH = 256
SHIFT = 14


def _geom(n_nodes, E):
    R = (-(-n_nodes // NW) + 7) // 8 * 8
    WSTRIDE = E + FLUSH
    return R, NW * R, WSTRIDE


def _prep_sc(src, dst, n_nodes):
    E = src.shape[0]
    R, _, WSTRIDE = _geom(n_nodes, E)
    DUMP = R
    NCHUNK = E // CHUNK
    assert NCHUNK * CHUNK == E and NCHUNK % 2 == 0
    NGRP = CHUNK // L
    MBUF = CHUNK + FLUSH + L

    @functools.partial(
        pl.kernel,
        out_type=(jax.ShapeDtypeStruct((NW * WSTRIDE,), I32),
                  jax.ShapeDtypeStruct((NW * L,), I32)),
        mesh=_MESH,
        scratch_types=[
            pltpu.VMEM((CHUNK,), I32), pltpu.VMEM((CHUNK,), I32),
            pltpu.VMEM((CHUNK,), I32), pltpu.VMEM((CHUNK,), I32),
            pltpu.VMEM((MBUF,), I32),
            pltpu.VMEM((FLUSH,), I32), pltpu.VMEM((FLUSH,), I32),
            pltpu.VMEM((L,), I32),
            pltpu.SemaphoreType.DMA, pltpu.SemaphoreType.DMA,
            pltpu.SemaphoreType.DMA, pltpu.SemaphoreType.DMA,
        ],
        compiler_params=_SC_PARAMS,
    )
    def k(src_hbm, dst_hbm, plist_hbm, cnt_hbm, src_v0, src_v1, dst_v0,
          dst_v1, mbuf, fstage0, fstage1, cstage, sem_in0, sem_in1,
          sem_f0, sem_f1):
        wid = lax.axis_index("s") * 2 + lax.axis_index("c")
        lo = wid * R
        base_out = wid * WSTRIDE
        lane = lax.iota(I32, L)
        srcs = (src_v0, src_v1)
        dsts = (dst_v0, dst_v1)
        insems = (sem_in0, sem_in1)
        fstgs = (fstage0, fstage1)
        fsems = (sem_f0, sem_f1)

        def load_chunk(ci, s):
            pltpu.async_copy(src_hbm.at[pl.ds(ci * CHUNK, CHUNK)],
                             srcs[s], insems[s])
            pltpu.async_copy(dst_hbm.at[pl.ds(ci * CHUNK, CHUNK)],
                             dsts[s], insems[s])

        def wait_chunk(s):
            pltpu.make_async_copy(src_hbm.at[pl.ds(0, CHUNK)],
                                  srcs[s], insems[s]).wait()
            pltpu.make_async_copy(dst_hbm.at[pl.ds(0, CHUNK)],
                                  dsts[s], insems[s]).wait()

        def flush(pos, nf):
            def flush_slot(s):
                stg = fstgs[s]

                @pl.when(nf >= 2)
                def _():
                    pltpu.make_async_copy(
                        plist_hbm.at[pl.ds(0, FLUSH)], stg,
                        fsems[s]).wait()
                for t in range(FLUSH // L):
                    stg[pl.ds(t * L, L)] = mbuf[pl.ds(t * L, L)]
                pltpu.async_copy(
                    stg,
                    plist_hbm.at[pl.ds(base_out + nf * FLUSH, FLUSH)],
                    fsems[s])

            @pl.when(nf % 2 == 0)
            def _():
                flush_slot(0)

            @pl.when(nf % 2 == 1)
            def _():
                flush_slot(1)
            for t in range(MBUF // L - FLUSH // L):
                mbuf[pl.ds(t * L, L)] = mbuf[pl.ds(FLUSH + t * L, L)]
            return pos - FLUSH

        def scan_one(ci, s, carry):
            pos, nf = carry
            wait_chunk(s)

            def group(gi, pos):
                s16 = srcs[s][pl.ds(gi * L, L)]
                dl = dsts[s][pl.ds(gi * L, L)] - lo
                mask = (dl >= 0) & (dl < R)
                pm = plsc.cumsum(mask.astype(I32))
                idx = pos + pm - 1
                plsc.store_scatter(mbuf, [idx],
                                   s16 | (dl << SHIFT), mask=mask)
                return pos + jnp.max(pm)

            pos = lax.fori_loop(0, NGRP, group, pos)

            @pl.when(ci + 2 < NCHUNK)
            def _():
                load_chunk(ci + 2, s)

            def do_flush(_, carry):
                pos, nf = carry
                return flush(pos, nf), nf + 1

            return lax.fori_loop(0, pos // FLUSH, do_flush, (pos, nf))

        load_chunk(0, 0)
        load_chunk(1, 1)

        def scan_pair(i, carry):
            carry = scan_one(2 * i, 0, carry)
            return scan_one(2 * i + 1, 1, carry)

        pos, nf = lax.fori_loop(0, NCHUNK // 2, scan_pair, (0, 0))

        padword = (wid * 8) | (DUMP << SHIFT)
        pad16 = jnp.full((L,), padword, I32)
        for t in range(FLUSH // L):
            plsc.store_scatter(mbuf, [pos + t * L + lane], pad16)
        flush(pos, nf)
        total = nf + 1

        cstage[...] = jnp.full((L,), total * FLUSH, I32)
        pltpu.sync_copy(cstage, cnt_hbm.at[pl.ds(wid * L, L)])
        pltpu.make_async_copy(plist_hbm.at[pl.ds(0, FLUSH)], fstage0,
                              sem_f0).wait()

        @pl.when(total >= 2)
        def _():
            pltpu.make_async_copy(plist_hbm.at[pl.ds(0, FLUSH)],
                                  fstage1, sem_f1).wait()

    return k(src, dst)


def _segmax_sc(hp, plist, counts, n_nodes, E):
    R, NPAD, WSTRIDE = _geom(n_nodes, E)
    RLOC = R + 8
    DP = D // 2

    @functools.partial(
        pl.kernel,
        out_type=jax.ShapeDtypeStruct((NPAD, DP), F32),
        mesh=_MESH,
        scratch_types=[
            pltpu.VMEM((GB,), I32), pltpu.VMEM((GB,), I32),
            pltpu.VMEM((GB,), I32), pltpu.VMEM((GB,), I32),
            pltpu.VMEM((GB,), I32), pltpu.VMEM((GB,), I32),
            pltpu.VMEM((GB, D), F32), pltpu.VMEM((GB, D), F32),
            pltpu.VMEM((RLOC, DP), F32),
            pltpu.VMEM((L,), I32),
            pltpu.SemaphoreType.DMA, pltpu.SemaphoreType.DMA,
            pltpu.SemaphoreType.DMA, pltpu.SemaphoreType.DMA,
        ],
        compiler_params=_SC_PARAMS,
    )
    def k(hp_hbm, plist_hbm, cnt_hbm, out_hbm, pbuf0, pbuf1, sidx0,
          sidx1, didx0, didx1, rows0, rows1, acc, cntv, sem_i0, sem_i1,
          sem_g0, sem_g1):
        wid = lax.axis_index("s") * 2 + lax.axis_index("c")
        lo = wid * R
        base = wid * WSTRIDE
        zero16 = jnp.zeros((L,), F32)
        lane = lax.iota(I32, L)
        pbufs = (pbuf0, pbuf1)
        sidxs = (sidx0, sidx1)
        didxs = (didx0, didx1)
        rowss = (rows0, rows1)
        isems = (sem_i0, sem_i1)
        gsems = (sem_g0, sem_g1)

        @pl.loop(0, RLOC)
        def _(r):
            for c in range(DP // L):
                acc[r, pl.ds(c * L, L)] = zero16

        pltpu.sync_copy(cnt_hbm.at[pl.ds(wid * L, L)], cntv)
        nb = jnp.max(cntv[...]) // GB

        def load_p(b, s):
            off = jnp.minimum(b, nb - 1) * GB
            pltpu.async_copy(plist_hbm.at[pl.ds(base + off, GB)],
                             pbufs[s], isems[s])

        def wait_p(s):
            pltpu.make_async_copy(plist_hbm.at[pl.ds(0, GB)], pbufs[s],
                                  isems[s]).wait()

        def decode(s):
            for t in range(GB // L):
                w = pbufs[s][pl.ds(t * L, L)]
                sidxs[s][pl.ds(t * L, L)] = w & ((1 << SHIFT) - 1)
                didxs[s][pl.ds(t * L, L)] = lax.shift_right_logical(
                    w, SHIFT)

        def gather(s):
            pltpu.async_copy(hp_hbm.at[sidxs[s]], rowss[s], gsems[s])

        def wait_g(s):
            pltpu.make_async_copy(hp_hbm.at[sidxs[s]], rowss[s],
                                  gsems[s]).wait()

        def rmw(s):
            @pl.loop(0, GB // L)
            def _(g):
                dv = didxs[s][pl.ds(g * L, L)]
                for l in range(L):
                    d = jnp.max(jnp.where(lane == l, dv, 0))
                    j = g * L + l
                    for c in range(DP // L):
                        sl = pl.ds(c * L, L)
                        a = plsc.bitcast(acc[d, sl], BF16)
                        r = plsc.bitcast(rowss[s][j, sl], BF16)
                        acc[d, sl] = plsc.bitcast(jnp.maximum(a, r), F32)

        pltpu.sync_copy(plist_hbm.at[pl.ds(base, GB)], pbuf0)
        decode(0)
        gather(0)
        load_p(1, 1)

        def step(i, carry):
            b = 2 * i
            wait_p(1)
            decode(1)
            load_p(b + 2, 0)
            wait_g(0)
            gather(1)
            rmw(0)
            wait_p(0)
            decode(0)
            load_p(b + 3, 1)
            wait_g(1)
            gather(0)
            rmw(1)
            return carry

        lax.fori_loop(0, nb // 2, step, 0)
        pltpu.make_async_copy(plist_hbm.at[pl.ds(0, GB)], pbuf1,
                              sem_i1).wait()
        wait_g(0)

        pltpu.sync_copy(acc.at[pl.ds(0, R)], out_hbm.at[pl.ds(lo, R)])

    return k(hp, plist, counts)


def _lin_relu(x, W, b):
    M, K = x.shape
    Kn = W.shape[1]
    BM = 1000

    def body(x_ref, w_ref, b_ref, o_ref):
        o_ref[...] = jnp.maximum(
            jnp.dot(x_ref[...], w_ref[...], preferred_element_type=F32)
            + b_ref[...], 0.0).astype(BF16)

    return pl.pallas_call(
        body,
        grid=(M // BM,),
        in_specs=[
            pl.BlockSpec((BM, K), lambda i: (i, 0)),
            pl.BlockSpec((K, Kn), lambda i: (0, 0)),
            pl.BlockSpec((1, Kn), lambda i: (0, 0)),
        ],
        out_specs=pl.BlockSpec((BM, Kn), lambda i: (i, 0)),
        out_shape=jax.ShapeDtypeStruct((M, Kn), BF16),
    )(x, W, b.reshape(1, Kn))


def _mid(x, hn, Ws, Wn, b, Wp, bp):
    M, K = x.shape
    BM = 1000

    def body(x_ref, hn_ref, ws_ref, wn_ref, b_ref, wp_ref, bp_ref,
             h1_ref, hp1_ref):
        t = (jnp.dot(x_ref[...], ws_ref[...], preferred_element_type=F32)
             + jnp.dot(hn_ref[...], wn_ref[...], preferred_element_type=F32)
             + b_ref[...])
        t = jnp.maximum(t, 0.0)
        nrm = jnp.sqrt(jnp.sum(t * t, axis=1, keepdims=True))
        h1 = t / jnp.maximum(nrm, 1e-12)
        h1_ref[...] = h1
        hp1_ref[...] = jnp.maximum(
            jnp.dot(h1, wp_ref[...], preferred_element_type=F32)
            + bp_ref[...], 0.0).astype(BF16)

    return pl.pallas_call(
        body,
        grid=(M // BM,),
        in_specs=[
            pl.BlockSpec((BM, K), lambda i: (i, 0)),
            pl.BlockSpec((BM, K), lambda i: (i, 0)),
            pl.BlockSpec((K, K), lambda i: (0, 0)),
            pl.BlockSpec((K, K), lambda i: (0, 0)),
            pl.BlockSpec((1, K), lambda i: (0, 0)),
            pl.BlockSpec((K, K), lambda i: (0, 0)),
            pl.BlockSpec((1, K), lambda i: (0, 0)),
        ],
        out_specs=[
            pl.BlockSpec((BM, K), lambda i: (i, 0)),
            pl.BlockSpec((BM, K), lambda i: (i, 0)),
        ],
        out_shape=[
            jax.ShapeDtypeStruct((M, K), F32),
            jax.ShapeDtypeStruct((M, K), BF16),
        ],
    )(x, hn, Ws, Wn, b.reshape(1, K), Wp, bp.reshape(1, K))


def _final(h1, hn, Ws, Wn, b):
    M, K = h1.shape
    C = Ws.shape[1]

    Wsp = jnp.zeros((K, D), F32).at[:, :C].set(Ws)
    Wnp = jnp.zeros((K, D), F32).at[:, :C].set(Wn)
    bp = jnp.zeros((1, D), F32).at[0, :C].set(b)
    BM = 1000

    def body(h_ref, hn_ref, ws_ref, wn_ref, b_ref, o_ref):
        o_ref[...] = (
            jnp.dot(h_ref[...], ws_ref[...], preferred_element_type=F32)
            + jnp.dot(hn_ref[...], wn_ref[...], preferred_element_type=F32)
            + b_ref[...])

    out = pl.pallas_call(
        body,
        grid=(M // BM,),
        in_specs=[
            pl.BlockSpec((BM, K), lambda i: (i, 0)),
            pl.BlockSpec((BM, K), lambda i: (i, 0)),
            pl.BlockSpec((K, D), lambda i: (0, 0)),
            pl.BlockSpec((K, D), lambda i: (0, 0)),
            pl.BlockSpec((1, D), lambda i: (0, 0)),
        ],
        out_specs=pl.BlockSpec((BM, D), lambda i: (i, 0)),
        out_shape=jax.ShapeDtypeStruct((M, D), F32),
    )(h1, hn, Wsp, Wnp, bp)
    return out[:, :C]


def kernel(x, edge_index, Wp0, bp0, Ws0, Wn0, b0, Wp1, bp1, Ws1, Wn1, b1):
    n = x.shape[0]
    E = edge_index.shape[1]
    src = edge_index[0].astype(I32)
    dst = edge_index[1].astype(I32)

    def pack(a):
        p = lax.bitcast_convert_type(a.reshape(n, D // 2, 2), F32)
        return jnp.concatenate([p, jnp.zeros((n, D - D // 2), F32)], axis=1)

    def unpack(p):
        return lax.bitcast_convert_type(p, BF16).reshape(-1, D)[:n]

    plist, counts = _prep_sc(src, dst, n)
    hp0 = _lin_relu(x, Wp0, bp0)
    hn0 = unpack(_segmax_sc(pack(hp0), plist, counts, n, E))
    h1, hp1 = _mid(x, hn0, Ws0, Wn0, b0, Wp1, bp1)
    hn1 = unpack(_segmax_sc(pack(hp1), plist, counts, n, E))
    return _final(h1, hn1, Ws1, Wn1, b1)


# speedup vs baseline: 4.0260x; 1.0212x over previous
import functools

import jax
import jax.numpy as jnp
from jax import lax
from jax.experimental import pallas as pl
from jax.experimental.pallas import tpu as pltpu
from jax.experimental.pallas import tpu_sc as plsc

F32 = jnp.float32
I32 = jnp.int32
BF16 = jnp.bfloat16

NW = 32
L = 16
D = 128
CHUNK = 2000
GB = 128

_MESH = plsc.VectorSubcoreMesh(core_axis_name="c", subcore_axis_name="s")
_SC_PARAMS = pltpu.CompilerParams(needs_layout_passes=False)
FLUSH = 256
SHIFT = 14


def _geom(n_nodes, E):
    R = (-(-n_nodes // NW) + 7) // 8 * 8
    WSTRIDE = E + FLUSH
    return R, NW * R, WSTRIDE


def _prep_sc(src, dst, n_nodes):
    E = src.shape[0]
    R, _, WSTRIDE = _geom(n_nodes, E)
    DUMP = R
    NCHUNK = E // CHUNK
    assert NCHUNK * CHUNK == E and NCHUNK % 2 == 0
    NGRP = CHUNK // L
    MBUF = CHUNK + FLUSH + L

    @functools.partial(
        pl.kernel,
        out_type=(jax.ShapeDtypeStruct((NW * WSTRIDE,), I32),
                  jax.ShapeDtypeStruct((NW * L,), I32)),
        mesh=_MESH,
        scratch_types=[
            pltpu.VMEM((CHUNK,), I32), pltpu.VMEM((CHUNK,), I32),
            pltpu.VMEM((CHUNK,), I32), pltpu.VMEM((CHUNK,), I32),
            pltpu.VMEM((MBUF,), I32),
            pltpu.VMEM((FLUSH,), I32), pltpu.VMEM((FLUSH,), I32),
            pltpu.VMEM((L,), I32),
            pltpu.SemaphoreType.DMA, pltpu.SemaphoreType.DMA,
            pltpu.SemaphoreType.DMA, pltpu.SemaphoreType.DMA,
        ],
        compiler_params=_SC_PARAMS,
    )
    def k(src_hbm, dst_hbm, plist_hbm, cnt_hbm, src_v0, src_v1, dst_v0,
          dst_v1, mbuf, fstage0, fstage1, cstage, sem_in0, sem_in1,
          sem_f0, sem_f1):
        wid = lax.axis_index("s") * 2 + lax.axis_index("c")
        lo = wid * R
        base_out = wid * WSTRIDE
        lane = lax.iota(I32, L)
        srcs = (src_v0, src_v1)
        dsts = (dst_v0, dst_v1)
        insems = (sem_in0, sem_in1)
        fstgs = (fstage0, fstage1)
        fsems = (sem_f0, sem_f1)

        def load_chunk(ci, s):
            pltpu.async_copy(src_hbm.at[pl.ds(ci * CHUNK, CHUNK)],
                             srcs[s], insems[s])
            pltpu.async_copy(dst_hbm.at[pl.ds(ci * CHUNK, CHUNK)],
                             dsts[s], insems[s])

        def wait_chunk(s):
            pltpu.make_async_copy(src_hbm.at[pl.ds(0, CHUNK)],
                                  srcs[s], insems[s]).wait()
            pltpu.make_async_copy(dst_hbm.at[pl.ds(0, CHUNK)],
                                  dsts[s], insems[s]).wait()

        def flush(pos, nf):
            def flush_slot(s):
                stg = fstgs[s]

                @pl.when(nf >= 2)
                def _():
                    pltpu.make_async_copy(
                        plist_hbm.at[pl.ds(0, FLUSH)], stg,
                        fsems[s]).wait()
                for t in range(FLUSH // L):
                    stg[pl.ds(t * L, L)] = mbuf[pl.ds(t * L, L)]
                pltpu.async_copy(
                    stg,
                    plist_hbm.at[pl.ds(base_out + nf * FLUSH, FLUSH)],
                    fsems[s])

            @pl.when(nf % 2 == 0)
            def _():
                flush_slot(0)

            @pl.when(nf % 2 == 1)
            def _():
                flush_slot(1)
            for t in range(MBUF // L - FLUSH // L):
                mbuf[pl.ds(t * L, L)] = mbuf[pl.ds(FLUSH + t * L, L)]
            return pos - FLUSH

        def scan_one(ci, s, carry):
            pos, nf = carry
            wait_chunk(s)

            def group(gi, pos):
                s16 = srcs[s][pl.ds(gi * L, L)]
                d16 = dsts[s][pl.ds(gi * L, L)]
                t = (s16 | (d16 << SHIFT)) - (lo << SHIFT)
                mask = t.astype(jnp.uint32) < jnp.uint32(R << SHIFT)
                pm = plsc.cumsum(mask.astype(I32))
                plsc.store_scatter(mbuf, [pm + (pos - 1)], t, mask=mask)
                return pos + pm[L - 1]

            pos = lax.fori_loop(0, NGRP, group, pos)

            @pl.when(ci + 2 < NCHUNK)
            def _():
                load_chunk(ci + 2, s)

            def do_flush(_, carry):
                pos, nf = carry
                return flush(pos, nf), nf + 1

            return lax.fori_loop(0, pos // FLUSH, do_flush, (pos, nf))

        load_chunk(0, 0)
        load_chunk(1, 1)

        def scan_pair(i, carry):
            carry = scan_one(2 * i, 0, carry)
            return scan_one(2 * i + 1, 1, carry)

        pos, nf = lax.fori_loop(0, NCHUNK // 2, scan_pair, (0, 0))

        padword = (wid * 8) | (DUMP << SHIFT)
        pad16 = jnp.full((L,), padword, I32)
        for t in range(FLUSH // L):
            plsc.store_scatter(mbuf, [pos + t * L + lane], pad16)
        flush(pos, nf)
        total = nf + 1

        cstage[...] = jnp.full((L,), total * FLUSH, I32)
        pltpu.sync_copy(cstage, cnt_hbm.at[pl.ds(wid * L, L)])
        pltpu.make_async_copy(plist_hbm.at[pl.ds(0, FLUSH)], fstage0,
                              sem_f0).wait()

        @pl.when(total >= 2)
        def _():
            pltpu.make_async_copy(plist_hbm.at[pl.ds(0, FLUSH)],
                                  fstage1, sem_f1).wait()

    return k(src, dst)


def _segmax_sc(hp, plist, counts, n_nodes, E):
    R, NPAD, WSTRIDE = _geom(n_nodes, E)
    RLOC = R + 8
    DP = D // 2

    @functools.partial(
        pl.kernel,
        out_type=jax.ShapeDtypeStruct((NPAD, DP), F32),
        mesh=_MESH,
        scratch_types=[
            pltpu.VMEM((GB,), I32), pltpu.VMEM((GB,), I32),
            pltpu.VMEM((GB,), I32), pltpu.VMEM((GB,), I32),
            pltpu.VMEM((GB,), I32), pltpu.VMEM((GB,), I32),
            pltpu.VMEM((GB, D), F32), pltpu.VMEM((GB, D), F32),
            pltpu.VMEM((RLOC, DP), F32),
            pltpu.VMEM((L,), I32),
            pltpu.SemaphoreType.DMA, pltpu.SemaphoreType.DMA,
            pltpu.SemaphoreType.DMA, pltpu.SemaphoreType.DMA,
        ],
        compiler_params=_SC_PARAMS,
    )
    def k(hp_hbm, plist_hbm, cnt_hbm, out_hbm, pbuf0, pbuf1, sidx0,
          sidx1, didx0, didx1, rows0, rows1, acc, cntv, sem_i0, sem_i1,
          sem_g0, sem_g1):
        wid = lax.axis_index("s") * 2 + lax.axis_index("c")
        lo = wid * R
        base = wid * WSTRIDE
        zero16 = jnp.zeros((L,), F32)
        lane = lax.iota(I32, L)
        pbufs = (pbuf0, pbuf1)
        sidxs = (sidx0, sidx1)
        didxs = (didx0, didx1)
        rowss = (rows0, rows1)
        isems = (sem_i0, sem_i1)
        gsems = (sem_g0, sem_g1)

        @pl.loop(0, RLOC)
        def _(r):
            for c in range(DP // L):
                acc[r, pl.ds(c * L, L)] = zero16

        pltpu.sync_copy(cnt_hbm.at[pl.ds(wid * L, L)], cntv)
        nb = jnp.max(cntv[...]) // GB

        def load_p(b, s):
            off = jnp.minimum(b, nb - 1) * GB
            pltpu.async_copy(plist_hbm.at[pl.ds(base + off, GB)],
                             pbufs[s], isems[s])

        def wait_p(s):
            pltpu.make_async_copy(plist_hbm.at[pl.ds(0, GB)], pbufs[s],
                                  isems[s]).wait()

        def decode(s):
            for t in range(GB // L):
                w = pbufs[s][pl.ds(t * L, L)]
                sidxs[s][pl.ds(t * L, L)] = w & ((1 << SHIFT) - 1)
                didxs[s][pl.ds(t * L, L)] = lax.shift_right_logical(
                    w, SHIFT)

        def gather(s):
            pltpu.async_copy(hp_hbm.at[sidxs[s]], rowss[s], gsems[s])

        def wait_g(s):
            pltpu.make_async_copy(hp_hbm.at[sidxs[s]], rowss[s],
                                  gsems[s]).wait()

        def rmw(s):
            @pl.loop(0, GB // L)
            def _(g):
                dv = didxs[s][pl.ds(g * L, L)]
                for l in range(L):
                    d = dv[l]
                    j = g * L + l
                    for c in range(DP // L):
                        sl = pl.ds(c * L, L)
                        a = plsc.bitcast(acc[d, sl], BF16)
                        r = plsc.bitcast(rowss[s][j, sl], BF16)
                        acc[d, sl] = plsc.bitcast(jnp.maximum(a, r), F32)

        pltpu.sync_copy(plist_hbm.at[pl.ds(base, GB)], pbuf0)
        decode(0)
        gather(0)
        load_p(1, 1)

        def step(i, carry):
            b = 2 * i
            wait_p(1)
            decode(1)
            load_p(b + 2, 0)
            wait_g(0)
            gather(1)
            rmw(0)
            wait_p(0)
            decode(0)
            load_p(b + 3, 1)
            wait_g(1)
            gather(0)
            rmw(1)
            return carry

        lax.fori_loop(0, nb // 2, step, 0)
        pltpu.make_async_copy(plist_hbm.at[pl.ds(0, GB)], pbuf1,
                              sem_i1).wait()
        wait_g(0)

        pltpu.sync_copy(acc.at[pl.ds(0, R)], out_hbm.at[pl.ds(lo, R)])

    return k(hp, plist, counts)


def _lin_relu(x, W, b):
    M, K = x.shape
    Kn = W.shape[1]
    BM = 1000

    def body(x_ref, w_ref, b_ref, o_ref):
        o_ref[...] = jnp.maximum(
            jnp.dot(x_ref[...], w_ref[...], preferred_element_type=F32)
            + b_ref[...], 0.0).astype(BF16)

    return pl.pallas_call(
        body,
        grid=(M // BM,),
        in_specs=[
            pl.BlockSpec((BM, K), lambda i: (i, 0)),
            pl.BlockSpec((K, Kn), lambda i: (0, 0)),
            pl.BlockSpec((1, Kn), lambda i: (0, 0)),
        ],
        out_specs=pl.BlockSpec((BM, Kn), lambda i: (i, 0)),
        out_shape=jax.ShapeDtypeStruct((M, Kn), BF16),
    )(x, W, b.reshape(1, Kn))


def _mid(x, hn, Ws, Wn, b, Wp, bp):
    M, K = x.shape
    BM = 1000

    def body(x_ref, hn_ref, ws_ref, wn_ref, b_ref, wp_ref, bp_ref,
             h1_ref, hp1_ref):
        t = (jnp.dot(x_ref[...], ws_ref[...], preferred_element_type=F32)
             + jnp.dot(hn_ref[...], wn_ref[...], preferred_element_type=F32)
             + b_ref[...])
        t = jnp.maximum(t, 0.0)
        nrm = jnp.sqrt(jnp.sum(t * t, axis=1, keepdims=True))
        h1 = t / jnp.maximum(nrm, 1e-12)
        h1_ref[...] = h1
        hp1_ref[...] = jnp.maximum(
            jnp.dot(h1, wp_ref[...], preferred_element_type=F32)
            + bp_ref[...], 0.0).astype(BF16)

    return pl.pallas_call(
        body,
        grid=(M // BM,),
        in_specs=[
            pl.BlockSpec((BM, K), lambda i: (i, 0)),
            pl.BlockSpec((BM, K), lambda i: (i, 0)),
            pl.BlockSpec((K, K), lambda i: (0, 0)),
            pl.BlockSpec((K, K), lambda i: (0, 0)),
            pl.BlockSpec((1, K), lambda i: (0, 0)),
            pl.BlockSpec((K, K), lambda i: (0, 0)),
            pl.BlockSpec((1, K), lambda i: (0, 0)),
        ],
        out_specs=[
            pl.BlockSpec((BM, K), lambda i: (i, 0)),
            pl.BlockSpec((BM, K), lambda i: (i, 0)),
        ],
        out_shape=[
            jax.ShapeDtypeStruct((M, K), F32),
            jax.ShapeDtypeStruct((M, K), BF16),
        ],
    )(x, hn, Ws, Wn, b.reshape(1, K), Wp, bp.reshape(1, K))


def _final(h1, hn, Ws, Wn, b):
    M, K = h1.shape
    C = Ws.shape[1]

    Wsp = jnp.zeros((K, D), F32).at[:, :C].set(Ws)
    Wnp = jnp.zeros((K, D), F32).at[:, :C].set(Wn)
    bp = jnp.zeros((1, D), F32).at[0, :C].set(b)
    BM = 1000

    def body(h_ref, hn_ref, ws_ref, wn_ref, b_ref, o_ref):
        o_ref[...] = (
            jnp.dot(h_ref[...], ws_ref[...], preferred_element_type=F32)
            + jnp.dot(hn_ref[...], wn_ref[...], preferred_element_type=F32)
            + b_ref[...])

    out = pl.pallas_call(
        body,
        grid=(M // BM,),
        in_specs=[
            pl.BlockSpec((BM, K), lambda i: (i, 0)),
            pl.BlockSpec((BM, K), lambda i: (i, 0)),
            pl.BlockSpec((K, D), lambda i: (0, 0)),
            pl.BlockSpec((K, D), lambda i: (0, 0)),
            pl.BlockSpec((1, D), lambda i: (0, 0)),
        ],
        out_specs=pl.BlockSpec((BM, D), lambda i: (i, 0)),
        out_shape=jax.ShapeDtypeStruct((M, D), F32),
    )(h1, hn, Wsp, Wnp, bp)
    return out[:, :C]


def kernel(x, edge_index, Wp0, bp0, Ws0, Wn0, b0, Wp1, bp1, Ws1, Wn1, b1):
    n = x.shape[0]
    E = edge_index.shape[1]
    src = edge_index[0].astype(I32)
    dst = edge_index[1].astype(I32)

    def pack(a):
        p = lax.bitcast_convert_type(a.reshape(n, D // 2, 2), F32)
        return jnp.concatenate([p, jnp.zeros((n, D - D // 2), F32)], axis=1)

    def unpack(p):
        return lax.bitcast_convert_type(p, BF16).reshape(-1, D)[:n]

    plist, counts = _prep_sc(src, dst, n)
    hp0 = _lin_relu(x, Wp0, bp0)
    hn0 = unpack(_segmax_sc(pack(hp0), plist, counts, n, E))
    h1, hp1 = _mid(x, hn0, Ws0, Wn0, b0, Wp1, bp1)
    hn1 = unpack(_segmax_sc(pack(hp1), plist, counts, n, E))
    return _final(h1, hn1, Ws1, Wn1, b1)
